# Initial kernel scaffold; baseline (speedup 1.0000x reference)
import jax
import jax.numpy as jnp
from jax.experimental import pallas as pl


def kernel(h, edge_index, W1, b1, W2, b2):
    raise NotImplementedError("write your pallas kernel here")


# speedup vs baseline: 2.3944x; 2.3944x over previous
import functools

import jax
import jax.numpy as jnp
from jax import lax
from jax.experimental import pallas as pl
from jax.experimental.pallas import tpu as pltpu
from jax.experimental.pallas import tpu_sc as plsc

N = 10000
E = 320000
D = 128


_ROWS = 1000


def _tc_tables_body(h_ref, w1a_ref, w1b_ref, b1_ref, a_ref, b_ref):
    hblk = h_ref[...]
    dn = (((1,), (1,)), ((), ()))
    a_ref[...] = (
        lax.dot_general(hblk, w1a_ref[...], dn,
                        preferred_element_type=jnp.float32,
                        precision=lax.Precision.HIGHEST)
        + b1_ref[...]
    )
    b_ref[...] = lax.dot_general(hblk, w1b_ref[...], dn,
                                 preferred_element_type=jnp.float32,
                                 precision=lax.Precision.HIGHEST)


def _build_tables(h, w1a, w1b, b1):
    grid = N // _ROWS
    return pl.pallas_call(
        _tc_tables_body,
        grid=(grid,),
        in_specs=[
            pl.BlockSpec((_ROWS, D), lambda i: (i, 0)),
            pl.BlockSpec((D, D), lambda i: (0, 0)),
            pl.BlockSpec((D, D), lambda i: (0, 0)),
            pl.BlockSpec((1, D), lambda i: (0, 0)),
        ],
        out_specs=[
            pl.BlockSpec((_ROWS, D), lambda i: (i, 0)),
            pl.BlockSpec((_ROWS, D), lambda i: (i, 0)),
        ],
        out_shape=[
            jax.ShapeDtypeStruct((N, D), jnp.float32),
            jax.ShapeDtypeStruct((N, D), jnp.float32),
        ],
    )(h, w1a, w1b, b1)


_NC = 2
_NS = 16
_NW = _NC * _NS
_EW = E // _NW
_C = 80
_R = _EW // _C
_K = D // 16


def _sc_edge_body(a_hbm, b_hbm, src_hbm, dst_hbm, wv_hbm, tail_hbm, out_hbm,
                  srcv, dstv, arows, brows, outv, wv_v, tail_v, fold_v, sem):
    wid = lax.axis_index("s") * _NC + lax.axis_index("c")
    base = wid * _EW
    pltpu.sync_copy(wv_hbm, wv_v)
    pltpu.sync_copy(tail_hbm, tail_v)
    b2v = tail_v[...]
    w2c = [wv_v[pl.ds(16 * k, 16)] for k in range(_K)]

    def round_body(r, carry):
        off = base + r * _C
        pltpu.sync_copy(src_hbm.at[pl.ds(off, _C)], srcv)
        pltpu.sync_copy(dst_hbm.at[pl.ds(off, _C)], dstv)
        cp_a = pltpu.async_copy(a_hbm.at[srcv], arows, sem)
        cp_b = pltpu.async_copy(b_hbm.at[dstv], brows, sem)
        cp_a.wait()
        cp_b.wait()

        def edge_body(e, c):
            acc = b2v
            for k in range(_K):
                x = arows[e, pl.ds(16 * k, 16)] + brows[e, pl.ds(16 * k, 16)]
                acc = acc + jnp.maximum(x, 0.0) * w2c[k]
            for sh in (8, 4, 2, 1):
                fold_v[pl.ds(16, 16)] = acc
                acc = acc + fold_v[pl.ds(16 - sh, 16)]
            fold_v[pl.ds(16, 16)] = acc
            outv[pl.ds(e, 16)] = fold_v[pl.ds(31, 16)]
            return c

        lax.fori_loop(0, _C, edge_body, 0, unroll=4)
        pltpu.sync_copy(outv.at[pl.ds(0, _C)], out_hbm.at[pl.ds(off, _C)])
        return carry

    lax.fori_loop(0, _R, round_body, 0)


def _score_edges(a_tab, b_tab, src, dst, wv, tail):
    mesh = plsc.VectorSubcoreMesh(core_axis_name="c", subcore_axis_name="s")
    f = pl.kernel(
        _sc_edge_body,
        out_type=jax.ShapeDtypeStruct((E,), jnp.float32),
        mesh=mesh,
        scratch_types=[
            pltpu.VMEM((_C,), jnp.int32),
            pltpu.VMEM((_C,), jnp.int32),
            pltpu.VMEM((_C, D), jnp.float32),
            pltpu.VMEM((_C, D), jnp.float32),
            pltpu.VMEM((_C + 16,), jnp.float32),
            pltpu.VMEM((D,), jnp.float32),
            pltpu.VMEM((16,), jnp.float32),
            pltpu.VMEM((48,), jnp.float32),
            pltpu.SemaphoreType.DMA,
        ],
    )
    return f(a_tab, b_tab, src, dst, wv, tail)


def kernel(h, edge_index, W1, b1, W2, b2):
    w1a = W1[:, :D]
    w1b = W1[:, D:]
    b1r = b1.reshape(1, D)
    a_tab, b_tab = _build_tables(h, w1a, w1b, b1r)
    src = edge_index[0]
    dst = edge_index[1]
    wv = W2[0]
    tail = jnp.full((16,), b2[0], jnp.float32)
    return _score_edges(a_tab, b_tab, src, dst, wv, tail)
